# baseline (speedup 1.0000x reference)
import functools

import jax
import jax.numpy as jnp
from jax import lax
from jax.experimental import pallas as pl
from jax.experimental.pallas import tpu as pltpu
from jax.experimental.pallas import tpu_sc as plsc

N_NODES = 10000
N_EDGES = 320000
IN_DIM = 128
HID = 64

NC, NS = 2, 16
NW = NC * NS
CHUNK = 125
CW = 80
DEGW = 16
ROWS_PT = N_NODES // NS


def _sc_agg_body(with_deg, h_hbm, edges_hbm, z64_hbm, z16_hbm,
                 ones_hbm, agg_out, src_v, dst_v, rows0, rows1,
                 ones_v, agg_sh, deg_sh, h_sh, sem0, sem1):
    cid = lax.axis_index("c")
    sid = lax.axis_index("s")
    r0 = sid * ROWS_PT
    pltpu.sync_copy(h_hbm.at[pl.ds(r0, ROWS_PT)], h_sh.at[pl.ds(r0, ROWS_PT)])
    pltpu.sync_copy(z64_hbm.at[pl.ds(r0, ROWS_PT)], agg_sh.at[pl.ds(r0, ROWS_PT)])
    if with_deg:
        pltpu.sync_copy(z16_hbm.at[pl.ds(r0, ROWS_PT)], deg_sh.at[pl.ds(r0, ROWS_PT)])
        pltpu.sync_copy(ones_hbm, ones_v)
    wid = cid * NS + sid
    pltpu.sync_copy(edges_hbm.at[0, wid], src_v)
    pltpu.sync_copy(edges_hbm.at[1, wid], dst_v)
    plsc.subcore_barrier()

    def start_g(jj, buf, sem):
        pltpu.async_copy(h_sh.at[src_v.at[jj]], buf, sem)

    def wait_g(jj, buf, sem):
        pltpu.make_async_copy(h_sh.at[src_v.at[jj]], buf, sem).wait()

    start_g(0, rows0, sem0)
    start_g(1, rows1, sem1)

    def body(i, carry):
        j = 2 * i
        wait_g(j, rows0, sem0)
        pltpu.sync_copy(rows0, agg_sh.at[dst_v.at[j]], add=True)

        @pl.when(j + 2 < CW)
        def _():
            start_g(j + 2, rows0, sem0)

        wait_g(j + 1, rows1, sem1)
        pltpu.sync_copy(rows1, agg_sh.at[dst_v.at[j + 1]], add=True)

        @pl.when(j + 3 < CW)
        def _():
            start_g(j + 3, rows1, sem1)

        if with_deg:
            pltpu.sync_copy(ones_v, deg_sh.at[dst_v.at[j]], add=True)
            pltpu.sync_copy(ones_v, deg_sh.at[dst_v.at[j + 1]], add=True)
        return carry

    lax.fori_loop(0, CW // 2, body, 0)
    plsc.subcore_barrier()
    if with_deg:
        pltpu.sync_copy(agg_sh.at[pl.ds(r0, ROWS_PT)],
                        agg_out.at[cid, pl.ds(r0, ROWS_PT), pl.ds(0, HID)])
        pltpu.sync_copy(deg_sh.at[pl.ds(r0, ROWS_PT)],
                        agg_out.at[cid, pl.ds(r0, ROWS_PT), pl.ds(HID, DEGW)])
    else:
        pltpu.sync_copy(agg_sh.at[pl.ds(r0, ROWS_PT)],
                        agg_out.at[cid, pl.ds(r0, ROWS_PT)])


def _make_sc_agg(with_deg):
    ow = HID + DEGW if with_deg else HID
    out_type = [jax.ShapeDtypeStruct((NC, N_NODES, ow), jnp.float32)]
    scratch = [
        pltpu.VMEM((CW, CHUNK), jnp.int32),
        pltpu.VMEM((CW, CHUNK), jnp.int32),
        pltpu.VMEM((CHUNK, HID), jnp.float32),
        pltpu.VMEM((CHUNK, HID), jnp.float32),
        pltpu.VMEM((CHUNK, DEGW), jnp.float32),
        pltpu.VMEM_SHARED((N_NODES, HID), jnp.float32),
        pltpu.VMEM_SHARED((N_NODES, DEGW), jnp.float32),
        pltpu.VMEM_SHARED((N_NODES, HID), jnp.float32),
        pltpu.SemaphoreType.DMA,
        pltpu.SemaphoreType.DMA,
    ]
    def fn(h, edges, z64, z16, ones, agg_out, *scr):
        _sc_agg_body(with_deg, h, edges, z64, z16, ones, agg_out, *scr)

    return pl.kernel(
        fn,
        out_type=out_type,
        mesh=plsc.VectorSubcoreMesh(core_axis_name="c", subcore_axis_name="s",
                                    num_cores=NC, num_subcores=NS),
        scratch_types=scratch,
        compiler_params=pltpu.CompilerParams(use_tc_tiling_on_sc=False),
    )


_get_sc_agg = functools.cache(_make_sc_agg)

BP = 1000


def _tc_front_body(x_ref, wi_ref, bi_ref, wn_ref, ws_ref, bs_ref,
                   u_ref, t_ref):
    h = jnp.maximum(
        jnp.dot(x_ref[...], wi_ref[...], preferred_element_type=jnp.float32)
        + bi_ref[...], 0.0)
    u_ref[...] = jnp.dot(h, wn_ref[...], preferred_element_type=jnp.float32)
    t_ref[...] = jnp.dot(h, ws_ref[...],
                         preferred_element_type=jnp.float32) + bs_ref[...]


@jax.jit
def _tc_front(x, wi, bi, wn, ws, bs):
    full = pl.BlockSpec((HID, HID), lambda i: (0, 0))
    brow = pl.BlockSpec((1, HID), lambda i: (0, 0))
    blk = pl.BlockSpec((BP, HID), lambda i: (i, 0))
    return pl.pallas_call(
        _tc_front_body,
        grid=(N_NODES // BP,),
        in_specs=[pl.BlockSpec((BP, IN_DIM), lambda i: (i, 0)),
                  pl.BlockSpec((IN_DIM, HID), lambda i: (0, 0)),
                  brow, full, full, brow],
        out_specs=[blk, blk],
        out_shape=[jax.ShapeDtypeStruct((N_NODES, HID), jnp.float32),
                   jax.ShapeDtypeStruct((N_NODES, HID), jnp.float32)],
    )(x, wi, bi, wn, ws, bs)


def _tc_mid_body(t_ref, ad_ref, wn_ref, ws_ref, bs_ref,
                 u_ref, t2_ref, dg_ref):
    ad = ad_ref[0] + ad_ref[1]
    agg = ad[:, 0:HID]
    deg = jnp.maximum(ad[:, HID:HID + 1], 1.0)
    h = jnp.maximum(t_ref[...] + agg / deg, 0.0)
    u_ref[...] = jnp.dot(h, wn_ref[...], preferred_element_type=jnp.float32)
    t2_ref[...] = jnp.dot(h, ws_ref[...],
                          preferred_element_type=jnp.float32) + bs_ref[...]
    dg_ref[...] = jnp.broadcast_to(deg, (BP, DEGW))


@jax.jit
def _tc_mid(t, ad, wn, ws, bs):
    full = pl.BlockSpec((HID, HID), lambda i: (0, 0))
    brow = pl.BlockSpec((1, HID), lambda i: (0, 0))
    blk = pl.BlockSpec((BP, HID), lambda i: (i, 0))
    return pl.pallas_call(
        _tc_mid_body,
        grid=(N_NODES // BP,),
        in_specs=[blk,
                  pl.BlockSpec((NC, BP, HID + DEGW), lambda i: (0, i, 0)),
                  full, full, brow],
        out_specs=[blk, blk, pl.BlockSpec((BP, DEGW), lambda i: (i, 0))],
        out_shape=[jax.ShapeDtypeStruct((N_NODES, HID), jnp.float32),
                   jax.ShapeDtypeStruct((N_NODES, HID), jnp.float32),
                   jax.ShapeDtypeStruct((N_NODES, DEGW), jnp.float32)],
    )(t, ad, wn, ws, bs)


def _tc_final_body(t_ref, agg_ref, dg_ref, o_ref):
    agg = agg_ref[0] + agg_ref[1]
    deg = dg_ref[:, 0:1]
    o_ref[...] = jnp.maximum(t_ref[...] + agg / deg, 0.0)


@jax.jit
def _tc_final(t, agg, dg):
    blk = pl.BlockSpec((BP, HID), lambda i: (i, 0))
    return pl.pallas_call(
        _tc_final_body,
        grid=(N_NODES // BP,),
        in_specs=[blk,
                  pl.BlockSpec((NC, BP, HID), lambda i: (0, i, 0)),
                  pl.BlockSpec((BP, DEGW), lambda i: (i, 0))],
        out_specs=blk,
        out_shape=jax.ShapeDtypeStruct((N_NODES, HID), jnp.float32),
    )(t, agg, dg)


def kernel(x, edges, W_in, b_in, Ws0, bs0, Wn0, Ws1, bs1, Wn1):
    ed = edges.astype(jnp.int32).reshape(2, NW, CW, CHUNK)
    z64 = jnp.zeros((N_NODES, HID), jnp.float32)
    z16 = jnp.zeros((N_NODES, DEGW), jnp.float32)
    ones = jnp.ones((CHUNK, DEGW), jnp.float32)

    u0, t0 = _tc_front(x, W_in, b_in.reshape(1, HID), Wn0, Ws0,
                       bs0.reshape(1, HID))
    (ad0,) = _get_sc_agg(True)(u0, ed, z64, z16, ones)
    u1, t1, degs = _tc_mid(t0, ad0, Wn1, Ws1, bs1.reshape(1, HID))
    (ag1,) = _get_sc_agg(False)(u1, ed, z64, z16, ones)
    return _tc_final(t1, ag1, degs)
